# Initial kernel scaffold; baseline (speedup 1.0000x reference)
import jax
import jax.numpy as jnp
from jax.experimental import pallas as pl


def kernel(regime_idx, table):
    raise NotImplementedError("write your pallas kernel here")


# speedup vs baseline: 4.4612x; 4.4612x over previous
import functools

import jax
import jax.numpy as jnp
from jax import lax
from jax.experimental import pallas as pl
from jax.experimental.pallas import tpu as pltpu
from jax.experimental.pallas import tpu_sc as plsc

DIM = 64
B = 16384 * 26
NW = 32
BPW = B // NW
IDX_MINOR = 128
ROWS_PER_CHUNK = 512
GATHERS_PER_CHUNK = ROWS_PER_CHUNK // IDX_MINOR
N_CHUNKS = BPW // ROWS_PER_CHUNK
N_PAIRS = N_CHUNKS // 2
IDX_ROWS = BPW // IDX_MINOR


def _sc_gather(table, idx):
    mesh = plsc.VectorSubcoreMesh(core_axis_name="c", subcore_axis_name="s")

    @functools.partial(
        pl.kernel,
        out_type=jax.ShapeDtypeStruct((B, DIM), jnp.float32),
        mesh=mesh,
        compiler_params=pltpu.CompilerParams(use_tc_tiling_on_sc=False),
        scratch_types=[
            pltpu.VMEM((IDX_ROWS, IDX_MINOR), jnp.int32),
            pltpu.VMEM((ROWS_PER_CHUNK, DIM), jnp.float32),
            pltpu.VMEM((ROWS_PER_CHUNK, DIM), jnp.float32),
            pltpu.SemaphoreType.DMA,
            pltpu.SemaphoreType.DMA,
            pltpu.SemaphoreType.DMA,
            pltpu.SemaphoreType.DMA,
        ],
    )
    def k(table_hbm, idx_hbm, out_hbm, idx_v, rows0, rows1, g0, g1, o0, o1):
        wid = lax.axis_index("s") * 2 + lax.axis_index("c")
        base = wid * BPW

        pltpu.sync_copy(idx_hbm.at[wid], idx_v)

        def issue_gather(chunk, buf, gsem):
            for j in range(GATHERS_PER_CHUNK):
                pltpu.async_copy(
                    table_hbm.at[idx_v.at[chunk * GATHERS_PER_CHUNK + j]],
                    buf.at[pl.ds(j * IDX_MINOR, IDX_MINOR)],
                    gsem,
                )

        def wait_gather(buf, gsem):
            for j in range(GATHERS_PER_CHUNK):
                pltpu.make_async_copy(
                    table_hbm.at[idx_v.at[j]],
                    buf.at[pl.ds(j * IDX_MINOR, IDX_MINOR)],
                    gsem,
                ).wait()

        def issue_store(chunk, buf, osem):
            pltpu.async_copy(
                buf,
                out_hbm.at[pl.ds(base + chunk * ROWS_PER_CHUNK, ROWS_PER_CHUNK)],
                osem,
            )

        def wait_store(buf, osem):
            pltpu.make_async_copy(
                buf,
                out_hbm.at[pl.ds(base, ROWS_PER_CHUNK)],
                osem,
            ).wait()

        issue_gather(0, rows0, g0)

        def body(i, _):
            even = 2 * i
            odd = 2 * i + 1

            @pl.when(i > 0)
            def _():
                wait_store(rows1, o1)

            issue_gather(odd, rows1, g1)
            wait_gather(rows0, g0)
            issue_store(even, rows0, o0)

            @pl.when(i < N_PAIRS - 1)
            def _():
                wait_store(rows0, o0)
                issue_gather(even + 2, rows0, g0)

            wait_gather(rows1, g1)
            issue_store(odd, rows1, o1)
            return 0

        lax.fori_loop(0, N_PAIRS, body, 0)
        wait_store(rows0, o0)
        wait_store(rows1, o1)

    return k(table, idx)


def kernel(regime_idx, table):
    idx = regime_idx.reshape(NW, IDX_ROWS, IDX_MINOR).astype(jnp.int32)
    out = _sc_gather(table, idx)
    return out.reshape(regime_idx.shape[0], regime_idx.shape[1], DIM)
